# baseline (speedup 1.0000x reference)
import functools

import numpy as np

import jax
import jax.numpy as jnp
from jax import lax
from jax.experimental import pallas as pl
from jax.experimental.pallas import tpu as pltpu
from jax.experimental.pallas import tpu_sc as plsc

N = 10000
E = 320000
D = 128
H = 128
R = 4
G = 64
C = 16
L = 2

NC = 2
NS = 16
EPT = E // NS
BK = 800
SEGE = 4000
NSEG = EPT // SEGE
SEGB = SEGE // BK
CH = 2560
CHUNK = 128
STAGE = SEGE + CHUNK
ACC_ROWS = R * CH + 128
DUMP = R * CH
ZPT = ACC_ROWS // NS

_PERM = [0] * D
for _g in range(D // 32):
    for _l in range(16):
        _PERM[_g * 32 + 2 * _l] = _g * 32 + _l
        _PERM[_g * 32 + 2 * _l + 1] = _g * 32 + 16 + _l
_PERM_NP = np.array(_PERM, dtype=np.int32)
ROWS_T = CH // NS

_mesh = plsc.VectorSubcoreMesh(core_axis_name="c", subcore_axis_name="s")


@functools.partial(
    pl.kernel,
    mesh=_mesh,
    out_type=jax.ShapeDtypeStruct((R * N, D), jnp.float32),
    compiler_params=pltpu.CompilerParams(needs_layout_passes=False,
                                         use_tc_tiling_on_sc=False),
    scratch_types=[
        pltpu.VMEM((STAGE,), jnp.int32),
        pltpu.VMEM((STAGE,), jnp.int32),
        pltpu.VMEM((BK,), jnp.int32),
        pltpu.VMEM((BK,), jnp.int32),
        pltpu.VMEM((BK,), jnp.int32),
        pltpu.VMEM((CHUNK,), jnp.int32),
        pltpu.VMEM((CHUNK,), jnp.int32),
        pltpu.VMEM((CHUNK,), jnp.int32),
        pltpu.VMEM((CHUNK,), jnp.int32),
        pltpu.VMEM((CHUNK, D // 2), jnp.int32),
        pltpu.VMEM((CHUNK, D // 2), jnp.int32),
        pltpu.VMEM((CHUNK, D), jnp.float32),
        pltpu.VMEM((16, D), jnp.float32),
        pltpu.VMEM_SHARED((ACC_ROWS, D), jnp.float32),
        pltpu.SemaphoreType.DMA,
        pltpu.SemaphoreType.DMA,
    ],
)
def _sc_agg(x_hbm, src_hbm, dst_hbm, type_hbm, out_hbm,
            stage_src, stage_idx, srcb, dstb, typeb,
            chunk_srcA, chunk_idxA, chunk_srcB, chunk_idxB,
            rowsA, rowsB, rows_f, zbuf, acc, semA, semB):
    c = lax.axis_index("c")
    s = lax.axis_index("s")
    ebase = s * EPT

    def _zrow(i, _):
        for j in range(D // 16):
            zbuf[i, pl.ds(j * 16, 16)] = jnp.zeros((16,), jnp.float32)
        return 0
    lax.fori_loop(0, 16, _zrow, 0)

    for p in range(2):
        rng = c * 2 + p
        base = rng * CH

        def _zacc(j, _):
            pltpu.sync_copy(zbuf, acc.at[pl.ds(s * ZPT + j * 16, 16)])
            return 0
        lax.fori_loop(0, ZPT // 16, _zacc, 0)
        pltpu.sync_copy(zbuf.at[pl.ds(0, ZPT % 16)],
                        acc.at[pl.ds(s * ZPT + ZPT - ZPT % 16, ZPT % 16)])
        plsc.subcore_barrier()

        def _seg(g, _):
            def _blk(b, off):
                eoff = ebase + g * SEGE + b * BK
                pltpu.sync_copy(src_hbm.at[pl.ds(eoff, BK)], srcb)
                pltpu.sync_copy(dst_hbm.at[pl.ds(eoff, BK)], dstb)
                pltpu.sync_copy(type_hbm.at[pl.ds(eoff, BK)], typeb)

                def _vec(j, off):
                    dv = dstb[pl.ds(j * 16, 16)]
                    tv = typeb[pl.ds(j * 16, 16)]
                    sv = srcb[pl.ds(j * 16, 16)]
                    m = (dv >= base) & (dv < base + CH)
                    mi = jnp.where(m, 1, 0)
                    csum = jnp.cumsum(mi)
                    pos = off + csum - mi
                    iv = tv * CH + dv - base
                    plsc.store_scatter(stage_src, [pos], sv, mask=m)
                    plsc.store_scatter(stage_idx, [pos], iv, mask=m)
                    return off + csum[15]

                return lax.fori_loop(0, BK // 16, _vec, off)

            off = lax.fori_loop(0, SEGB, _blk, jnp.int32(0))

            for j in range(CHUNK // 16):
                stage_src[pl.ds(off + j * 16, 16)] = jnp.zeros((16,),
                                                               jnp.int32)
                stage_idx[pl.ds(off + j * 16, 16)] = jnp.full(
                    (16,), DUMP + s, jnp.int32)
            nch = jnp.maximum((off + CHUNK - 1) // CHUNK, 1)

            def _prep(k, csrc, cidx):
                for j in range(CHUNK // 16):
                    csrc[pl.ds(j * 16, 16)] = (
                        stage_src[pl.ds(k * CHUNK + j * 16, 16)])
                    cidx[pl.ds(j * 16, 16)] = (
                        stage_idx[pl.ds(k * CHUNK + j * 16, 16)])

            def _consume(rows_bf, cidx):
                def _row(r, _):
                    for g in range(D // 32):
                        iv = rows_bf[r, pl.ds(g * 16, 16)]
                        cv = plsc.bitcast(iv, jnp.bfloat16)
                        a, b = plsc.unpack(cv,
                                           format=plsc.PackFormat.INTERLEAVED)
                        rows_f[r, pl.ds(g * 32, 16)] = a
                        rows_f[r, pl.ds(g * 32 + 16, 16)] = b
                    return 0
                lax.fori_loop(0, CHUNK, _row, 0)
                pltpu.sync_copy(rows_f, acc.at[cidx], add=True)

            _prep(jnp.int32(0), chunk_srcA, chunk_idxA)
            HC = CHUNK // 2

            def _mk(csrc, rows, sem):
                return (
                    pltpu.make_async_copy(
                        x_hbm.at[csrc.at[pl.ds(0, HC)]],
                        rows.at[pl.ds(0, HC)], sem),
                    pltpu.make_async_copy(
                        x_hbm.at[csrc.at[pl.ds(HC, HC)]],
                        rows.at[pl.ds(HC, HC)], sem),
                )

            gA1, gA2 = _mk(chunk_srcA, rowsA, semA)
            gB1, gB2 = _mk(chunk_srcB, rowsB, semB)

            def _startA():
                gA1.start()
                gA2.start()

            def _startB():
                gB1.start()
                gB2.start()

            def _waitA():
                gA1.wait()
                gA2.wait()

            def _waitB():
                gB1.wait()
                gB2.wait()

            _startA()

            def _chunk(k, _):
                @pl.when(k % 2 == 1)
                def _():
                    _prep(k, chunk_srcB, chunk_idxB)
                    _startB()
                    _waitA()
                    _consume(rowsA, chunk_idxA)

                @pl.when(k % 2 == 0)
                def _():
                    _prep(k, chunk_srcA, chunk_idxA)
                    _startA()
                    _waitB()
                    _consume(rowsB, chunk_idxB)
                return 0
            lax.fori_loop(1, nch, _chunk, 0)

            @pl.when(nch % 2 == 1)
            def _():
                _waitA()
                _consume(rowsA, chunk_idxA)

            @pl.when(nch % 2 == 0)
            def _():
                _waitB()
                _consume(rowsB, chunk_idxB)
            return 0

        lax.fori_loop(0, NSEG, _seg, 0)
        plsc.subcore_barrier()

        tail = N - 3 * CH - 14 * ROWS_T
        for r in range(R):
            @pl.when((rng < 3) | (s < 14))
            def _():
                pltpu.sync_copy(
                    acc.at[pl.ds(r * CH + s * ROWS_T, ROWS_T)],
                    out_hbm.at[pl.ds(r * N + base + s * ROWS_T, ROWS_T)])

            @pl.when((rng == 3) & (s == 14))
            def _():
                pltpu.sync_copy(
                    acc.at[pl.ds(r * CH + 14 * ROWS_T, tail)],
                    out_hbm.at[pl.ds(r * N + base + 14 * ROWS_T, tail)])
        plsc.subcore_barrier()


BN = 1000


def _layer_body(scale_ref, x_ref, agg_ref, Wsl_ref, bsl_ref,
                W1_ref, b1_ref, W2_ref, b2_ref, o_ref):
    xb = x_ref[...]
    acc = jnp.dot(xb, Wsl_ref[...],
                  preferred_element_type=jnp.float32) + bsl_ref[...]
    for r in range(R):
        h = xb * scale_ref[0, r] + agg_ref[r]
        t = jnp.maximum(
            jnp.dot(h, W1_ref[r], preferred_element_type=jnp.float32)
            + b1_ref[r], 0.0)
        acc = acc + jnp.dot(t, W2_ref[r],
                            preferred_element_type=jnp.float32) + b2_ref[r]
    o_ref[...] = acc


def _tc_layer(scale_l, x, agg, Wsl_l, bsl_l, W1_l, b1_l, W2_l, b2_l):
    return pl.pallas_call(
        _layer_body,
        grid=(N // BN,),
        in_specs=[
            pl.BlockSpec(memory_space=pltpu.SMEM),
            pl.BlockSpec((BN, D), lambda i: (i, 0)),
            pl.BlockSpec((R, BN, D), lambda i: (0, i, 0)),
            pl.BlockSpec((D, H), lambda i: (0, 0)),
            pl.BlockSpec((1, H), lambda i: (0, 0)),
            pl.BlockSpec((R, D, H), lambda i: (0, 0, 0)),
            pl.BlockSpec((R, 1, H), lambda i: (0, 0, 0)),
            pl.BlockSpec((R, H, H), lambda i: (0, 0, 0)),
            pl.BlockSpec((R, 1, H), lambda i: (0, 0, 0)),
        ],
        out_specs=pl.BlockSpec((BN, H), lambda i: (i, 0)),
        out_shape=jax.ShapeDtypeStruct((N, H), jnp.float32),
    )(scale_l, x, agg, Wsl_l, bsl_l, W1_l, b1_l, W2_l, b2_l)


def _pool_body(batch_ref, x_ref, Wl1_ref, bl1_ref, Wl2_ref, bl2_ref, o_ref):
    bvals = batch_ref[...]
    gid = lax.broadcasted_iota(jnp.int32, (G, N), 0)
    oh = (bvals == gid).astype(jnp.float32)
    sums = jnp.dot(oh, x_ref[...], preferred_element_type=jnp.float32)
    cnt = jnp.sum(oh, axis=1, keepdims=True)
    pooled = sums / jnp.maximum(cnt, 1.0)
    h = jnp.maximum(
        jnp.dot(pooled, Wl1_ref[...], preferred_element_type=jnp.float32)
        + bl1_ref[...], 0.0)
    o_ref[...] = jnp.dot(h, Wl2_ref[...],
                         preferred_element_type=jnp.float32) + bl2_ref[...]


def _tc_pool(batch2d, x, Wl1, bl1, Wl2, bl2):
    return pl.pallas_call(
        _pool_body,
        out_shape=jax.ShapeDtypeStruct((G, C), jnp.float32),
    )(batch2d, x, Wl1, bl1, Wl2, bl2)


def kernel(x, edge_index, edge_type, batch, Wsl, bsl, eps, W1, b1, W2, b2,
           Wl1, bl1, Wl2, bl2):
    src = edge_index[0]
    dst = edge_index[1]
    scale = 1.0 + eps
    h = x
    for l in range(L):
        hb = h[:, _PERM_NP].astype(jnp.bfloat16)
        hbi = lax.bitcast_convert_type(hb.reshape(N, D // 2, 2), jnp.int32)
        agg = _sc_agg(hbi, src, dst, edge_type).reshape(R, N, H)
        h = _tc_layer(scale[l].reshape(1, R), h, agg,
                      Wsl[l], bsl[l].reshape(1, H),
                      W1[l], b1[l].reshape(R, 1, H),
                      W2[l], b2[l].reshape(R, 1, H))
    return _tc_pool(batch.reshape(1, N), h,
                    Wl1, bl1.reshape(1, H), Wl2, bl2.reshape(1, C))


# speedup vs baseline: 1.0227x; 1.0227x over previous
import functools

import numpy as np

import jax
import jax.numpy as jnp
from jax import lax
from jax.experimental import pallas as pl
from jax.experimental.pallas import tpu as pltpu
from jax.experimental.pallas import tpu_sc as plsc

N = 10000
E = 320000
D = 128
H = 128
R = 4
G = 64
C = 16
L = 2

NC = 2
NS = 16
EPT = E // NS
BK = 800
SEGE = 4000
NSEG = EPT // SEGE
SEGB = SEGE // BK
CH = 1280
CHUNK = 128
STAGE = SEGE + CHUNK
ACC_ROWS = R * CH + 128
DUMP = R * CH
ZPT = ACC_ROWS // NS
XPT = N // NS

_PERM = [0] * D
for _g in range(D // 32):
    for _l in range(16):
        _PERM[_g * 32 + 2 * _l] = _g * 32 + _l
        _PERM[_g * 32 + 2 * _l + 1] = _g * 32 + 16 + _l
_PERM_NP = np.array(_PERM, dtype=np.int32)
ROWS_T = CH // NS

_mesh = plsc.VectorSubcoreMesh(core_axis_name="c", subcore_axis_name="s")


@functools.partial(
    pl.kernel,
    mesh=_mesh,
    out_type=jax.ShapeDtypeStruct((R * N, D), jnp.float32),
    compiler_params=pltpu.CompilerParams(needs_layout_passes=False,
                                         use_tc_tiling_on_sc=False),
    scratch_types=[
        pltpu.VMEM((STAGE,), jnp.int32),
        pltpu.VMEM((STAGE,), jnp.int32),
        pltpu.VMEM((BK,), jnp.int32),
        pltpu.VMEM((BK,), jnp.int32),
        pltpu.VMEM((BK,), jnp.int32),
        pltpu.VMEM((CHUNK,), jnp.int32),
        pltpu.VMEM((CHUNK,), jnp.int32),
        pltpu.VMEM((CHUNK,), jnp.int32),
        pltpu.VMEM((CHUNK,), jnp.int32),
        pltpu.VMEM((CHUNK, D // 2), jnp.int32),
        pltpu.VMEM((CHUNK, D // 2), jnp.int32),
        pltpu.VMEM((CHUNK, D), jnp.float32),
        pltpu.VMEM((16, D), jnp.float32),
        pltpu.VMEM_SHARED((N, D // 2), jnp.int32),
        pltpu.VMEM_SHARED((ACC_ROWS, D), jnp.float32),
        pltpu.SemaphoreType.DMA,
        pltpu.SemaphoreType.DMA,
    ],
)
def _sc_agg(x_hbm, src_hbm, dst_hbm, type_hbm, out_hbm,
            stage_src, stage_idx, srcb, dstb, typeb,
            chunk_srcA, chunk_idxA, chunk_srcB, chunk_idxB,
            rowsA, rowsB, rows_f, zbuf, x_sp, acc, semA, semB):
    c = lax.axis_index("c")
    s = lax.axis_index("s")
    ebase = s * EPT

    def _zrow(i, _):
        for j in range(D // 16):
            zbuf[i, pl.ds(j * 16, 16)] = jnp.zeros((16,), jnp.float32)
        return 0
    lax.fori_loop(0, 16, _zrow, 0)

    pltpu.sync_copy(x_hbm.at[pl.ds(s * XPT, XPT)], x_sp.at[pl.ds(s * XPT, XPT)])

    for p in range(4):
        rng = c * 4 + p
        base = rng * CH

        def _zacc(j, _):
            pltpu.sync_copy(zbuf, acc.at[pl.ds(s * ZPT + j * 16, 16)])
            return 0
        lax.fori_loop(0, ZPT // 16, _zacc, 0)
        pltpu.sync_copy(zbuf.at[pl.ds(0, ZPT % 16)],
                        acc.at[pl.ds(s * ZPT + ZPT - ZPT % 16, ZPT % 16)])
        plsc.subcore_barrier()

        def _seg(g, _):
            def _blk(b, off):
                eoff = ebase + g * SEGE + b * BK
                pltpu.sync_copy(src_hbm.at[pl.ds(eoff, BK)], srcb)
                pltpu.sync_copy(dst_hbm.at[pl.ds(eoff, BK)], dstb)
                pltpu.sync_copy(type_hbm.at[pl.ds(eoff, BK)], typeb)

                def _vec(j, off):
                    dv = dstb[pl.ds(j * 16, 16)]
                    tv = typeb[pl.ds(j * 16, 16)]
                    sv = srcb[pl.ds(j * 16, 16)]
                    m = (dv >= base) & (dv < base + CH)
                    mi = jnp.where(m, 1, 0)
                    csum = jnp.cumsum(mi)
                    pos = off + csum - mi
                    iv = tv * CH + dv - base
                    plsc.store_scatter(stage_src, [pos], sv, mask=m)
                    plsc.store_scatter(stage_idx, [pos], iv, mask=m)
                    return off + csum[15]

                return lax.fori_loop(0, BK // 16, _vec, off)

            off = lax.fori_loop(0, SEGB, _blk, jnp.int32(0))

            for j in range(CHUNK // 16):
                stage_src[pl.ds(off + j * 16, 16)] = jnp.zeros((16,),
                                                               jnp.int32)
                stage_idx[pl.ds(off + j * 16, 16)] = jnp.full(
                    (16,), DUMP + s, jnp.int32)
            nch = jnp.maximum((off + CHUNK - 1) // CHUNK, 1)

            def _prep(k, csrc, cidx):
                for j in range(CHUNK // 16):
                    csrc[pl.ds(j * 16, 16)] = (
                        stage_src[pl.ds(k * CHUNK + j * 16, 16)])
                    cidx[pl.ds(j * 16, 16)] = (
                        stage_idx[pl.ds(k * CHUNK + j * 16, 16)])

            def _consume(rows_bf, cidx):
                def _row(r, _):
                    for g in range(D // 32):
                        iv = rows_bf[r, pl.ds(g * 16, 16)]
                        cv = plsc.bitcast(iv, jnp.bfloat16)
                        a, b = plsc.unpack(cv,
                                           format=plsc.PackFormat.INTERLEAVED)
                        rows_f[r, pl.ds(g * 32, 16)] = a
                        rows_f[r, pl.ds(g * 32 + 16, 16)] = b
                    return 0
                lax.fori_loop(0, CHUNK, _row, 0)
                pltpu.sync_copy(rows_f, acc.at[cidx], add=True)

            _prep(jnp.int32(0), chunk_srcA, chunk_idxA)
            HC = CHUNK // 2

            def _mk(csrc, rows, sem):
                return (
                    pltpu.make_async_copy(
                        x_sp.at[csrc.at[pl.ds(0, HC)]],
                        rows.at[pl.ds(0, HC)], sem),
                    pltpu.make_async_copy(
                        x_sp.at[csrc.at[pl.ds(HC, HC)]],
                        rows.at[pl.ds(HC, HC)], sem),
                )

            gA1, gA2 = _mk(chunk_srcA, rowsA, semA)
            gB1, gB2 = _mk(chunk_srcB, rowsB, semB)

            def _startA():
                gA1.start()
                gA2.start()

            def _startB():
                gB1.start()
                gB2.start()

            def _waitA():
                gA1.wait()
                gA2.wait()

            def _waitB():
                gB1.wait()
                gB2.wait()

            _startA()

            def _chunk(k, _):
                @pl.when(k % 2 == 1)
                def _():
                    _prep(k, chunk_srcB, chunk_idxB)
                    _startB()
                    _waitA()
                    _consume(rowsA, chunk_idxA)

                @pl.when(k % 2 == 0)
                def _():
                    _prep(k, chunk_srcA, chunk_idxA)
                    _startA()
                    _waitB()
                    _consume(rowsB, chunk_idxB)
                return 0
            lax.fori_loop(1, nch, _chunk, 0)

            @pl.when(nch % 2 == 1)
            def _():
                _waitA()
                _consume(rowsA, chunk_idxA)

            @pl.when(nch % 2 == 0)
            def _():
                _waitB()
                _consume(rowsB, chunk_idxB)
            return 0

        lax.fori_loop(0, NSEG, _seg, 0)
        plsc.subcore_barrier()

        for r in range(R):
            @pl.when((rng < 7) | (s < 13))
            def _():
                pltpu.sync_copy(
                    acc.at[pl.ds(r * CH + s * ROWS_T, ROWS_T)],
                    out_hbm.at[pl.ds(r * N + base + s * ROWS_T, ROWS_T)])
        plsc.subcore_barrier()


BN = 1000


def _layer_body(scale_ref, x_ref, agg_ref, Wsl_ref, bsl_ref,
                W1_ref, b1_ref, W2_ref, b2_ref, o_ref):
    xb = x_ref[...]
    acc = jnp.dot(xb, Wsl_ref[...],
                  preferred_element_type=jnp.float32) + bsl_ref[...]
    for r in range(R):
        h = xb * scale_ref[0, r] + agg_ref[r]
        t = jnp.maximum(
            jnp.dot(h, W1_ref[r], preferred_element_type=jnp.float32)
            + b1_ref[r], 0.0)
        acc = acc + jnp.dot(t, W2_ref[r],
                            preferred_element_type=jnp.float32) + b2_ref[r]
    o_ref[...] = acc


def _tc_layer(scale_l, x, agg, Wsl_l, bsl_l, W1_l, b1_l, W2_l, b2_l):
    return pl.pallas_call(
        _layer_body,
        grid=(N // BN,),
        in_specs=[
            pl.BlockSpec(memory_space=pltpu.SMEM),
            pl.BlockSpec((BN, D), lambda i: (i, 0)),
            pl.BlockSpec((R, BN, D), lambda i: (0, i, 0)),
            pl.BlockSpec((D, H), lambda i: (0, 0)),
            pl.BlockSpec((1, H), lambda i: (0, 0)),
            pl.BlockSpec((R, D, H), lambda i: (0, 0, 0)),
            pl.BlockSpec((R, 1, H), lambda i: (0, 0, 0)),
            pl.BlockSpec((R, H, H), lambda i: (0, 0, 0)),
            pl.BlockSpec((R, 1, H), lambda i: (0, 0, 0)),
        ],
        out_specs=pl.BlockSpec((BN, H), lambda i: (i, 0)),
        out_shape=jax.ShapeDtypeStruct((N, H), jnp.float32),
    )(scale_l, x, agg, Wsl_l, bsl_l, W1_l, b1_l, W2_l, b2_l)


def _pool_body(batch_ref, x_ref, Wl1_ref, bl1_ref, Wl2_ref, bl2_ref, o_ref):
    bvals = batch_ref[...]
    gid = lax.broadcasted_iota(jnp.int32, (G, N), 0)
    oh = (bvals == gid).astype(jnp.float32)
    sums = jnp.dot(oh, x_ref[...], preferred_element_type=jnp.float32)
    cnt = jnp.sum(oh, axis=1, keepdims=True)
    pooled = sums / jnp.maximum(cnt, 1.0)
    h = jnp.maximum(
        jnp.dot(pooled, Wl1_ref[...], preferred_element_type=jnp.float32)
        + bl1_ref[...], 0.0)
    o_ref[...] = jnp.dot(h, Wl2_ref[...],
                         preferred_element_type=jnp.float32) + bl2_ref[...]


def _tc_pool(batch2d, x, Wl1, bl1, Wl2, bl2):
    return pl.pallas_call(
        _pool_body,
        out_shape=jax.ShapeDtypeStruct((G, C), jnp.float32),
    )(batch2d, x, Wl1, bl1, Wl2, bl2)


def kernel(x, edge_index, edge_type, batch, Wsl, bsl, eps, W1, b1, W2, b2,
           Wl1, bl1, Wl2, bl2):
    src = edge_index[0]
    dst = edge_index[1]
    scale = 1.0 + eps
    h = x
    for l in range(L):
        hb = h[:, _PERM_NP].astype(jnp.bfloat16)
        hbi = lax.bitcast_convert_type(hb.reshape(N, D // 2, 2), jnp.int32)
        agg = _sc_agg(hbi, src, dst, edge_type).reshape(R, N, H)
        h = _tc_layer(scale[l].reshape(1, R), h, agg,
                      Wsl[l], bsl[l].reshape(1, H),
                      W1[l], b1[l].reshape(R, 1, H),
                      W2[l], b2[l].reshape(R, 1, H))
    return _tc_pool(batch.reshape(1, N), h,
                    Wl1, bl1.reshape(1, H), Wl2, bl2.reshape(1, C))
